# baseline (speedup 1.0000x reference)
import functools

import jax
import jax.numpy as jnp
from jax import lax
from jax.experimental import pallas as pl
from jax.experimental.pallas import tpu as pltpu
from jax.experimental.pallas import tpu_sc as plsc

_B, _N, _K = 32, 2048, 20
_C1P = 16
_C2P = 32
_R = 1024

_NW = 32
_GCHUNK = 128
_NBUF = 4
_M = _B * _N * _K
_MW = _M // _NW
_NCHUNK = _MW // _GCHUNK


def _sel_matrix(cp, c):
    rows = lax.broadcasted_iota(jnp.int32, (_K * cp, _K * c), 0)
    cols = lax.broadcasted_iota(jnp.int32, (_K * cp, _K * c), 1)
    k, r = rows // cp, rows % cp
    return jnp.where((r < c) & (cols == k * c + r), 1.0, 0.0).astype(
        jnp.float32
    )


def _prep_body(w1_ref, w2_ref, o1_ref, o2_ref):
    w1 = w1_ref[...]
    e1 = jnp.exp(w1 - jnp.max(w1, axis=0, keepdims=True))
    s1 = e1 / jnp.sum(e1, axis=0, keepdims=True)
    s1 = jnp.pad(s1, ((0, 0), (0, 2)))
    o1_ref[...] = jnp.dot(
        _sel_matrix(_C1P, 3), s1, preferred_element_type=jnp.float32
    )
    w2 = w2_ref[...]
    e2 = jnp.exp(w2 - jnp.max(w2, axis=0, keepdims=True))
    s2 = e2 / jnp.sum(e2, axis=0, keepdims=True)
    o2_ref[...] = jnp.dot(
        _sel_matrix(_C2P, 30), s2, preferred_element_type=jnp.float32
    )


def _prep_weights(W_fm1, W_fm2, interpret=False):
    return pl.pallas_call(
        _prep_body,
        out_shape=(
            jax.ShapeDtypeStruct((_K * _C1P, 32), jnp.float32),
            jax.ShapeDtypeStruct((_K * _C2P, 40), jnp.float32),
        ),
        interpret=interpret,
    )(W_fm1.reshape(_K * 3, 30), W_fm2.reshape(_K * 30, 40))


def _topk_body(xb_ref, xt_ref, idx_ref, xpad_ref):
    b = pl.program_id(0)
    xb = xb_ref[0]
    xt = xt_ref[0]
    xpad_ref[...] = jnp.pad(xb, ((0, 0), (0, _C1P - 3)))
    sqb = jnp.sum(xb * xb, axis=1, keepdims=True)
    sqf = jnp.sum(xt * xt, axis=0, keepdims=True)
    d = sqb + sqf - 2.0 * jnp.dot(xb, xt, preferred_element_type=jnp.float32)
    col = lax.broadcasted_iota(jnp.int32, (_R, _N), 1)
    key = (
        lax.bitcast_convert_type(jnp.maximum(d, 0.0), jnp.int32) & (-2048)
    ) | col
    int_min = jnp.int32(-(2**31))
    m = jnp.full((_R, 1), -1, jnp.int32)
    cols = []
    for _ in range(_K):
        s = m + (int_min + 1)
        v = key - s
        m = jnp.min(v, axis=1, keepdims=True) + s
        cols.append(m & 2047)
    idx_ref[0] = jnp.concatenate(cols, axis=1) + b * _N


def _topk(inputs, interpret=False):
    xt = jnp.transpose(inputs, (0, 2, 1))
    return pl.pallas_call(
        _topk_body,
        grid=(_B, _N // _R),
        in_specs=[
            pl.BlockSpec((1, _R, 3), lambda b, r: (b, r, 0)),
            pl.BlockSpec((1, 3, _N), lambda b, r: (b, 0, 0)),
        ],
        out_specs=[
            pl.BlockSpec((1, _R, _K), lambda b, r: (b, r, 0)),
            pl.BlockSpec((_R, _C1P), lambda b, r: (b * (_N // _R) + r, 0)),
        ],
        out_shape=[
            jax.ShapeDtypeStruct((_B, _N, _K), jnp.int32),
            jax.ShapeDtypeStruct((_B * _N, _C1P), jnp.float32),
        ],
        interpret=interpret,
    )(inputs, xt)


@functools.lru_cache(maxsize=None)
def _make_sc_gather(width):
    mesh = plsc.VectorSubcoreMesh(core_axis_name="c", subcore_axis_name="s")

    @functools.partial(
        pl.kernel,
        out_type=jax.ShapeDtypeStruct((_M, width), jnp.float32),
        mesh=mesh,
        scratch_types=[
            pltpu.VMEM((_NCHUNK, _GCHUNK), jnp.int32),
            pltpu.VMEM((_NBUF, _GCHUNK, width), jnp.float32),
        ] + [pltpu.SemaphoreType.DMA] * _NBUF,
        compiler_params=pltpu.CompilerParams(use_tc_tiling_on_sc=False),
    )
    def gather_kernel(table_hbm, idx_hbm, out_hbm, idxv, bufv, *sems):
        c = lax.axis_index("c")
        s = lax.axis_index("s")
        wid = s * 2 + c
        base = wid * _MW
        pltpu.sync_copy(idx_hbm.at[wid], idxv)

        def fire(chunk, b):
            pltpu.async_copy(table_hbm.at[idxv.at[chunk]], bufv.at[b], sems[b])

        def wait(b):
            pltpu.make_async_copy(
                table_hbm.at[idxv.at[0]], bufv.at[b], sems[b]
            ).wait()

        for b in range(_NBUF):
            fire(b, b)

        def outer(g, carry):
            for b in range(_NBUF):
                chunk = g * _NBUF + b
                wait(b)
                pltpu.sync_copy(
                    bufv.at[b],
                    out_hbm.at[pl.ds(base + chunk * _GCHUNK, _GCHUNK)],
                )

                @pl.when(chunk + _NBUF < _NCHUNK)
                def _():
                    fire(chunk + _NBUF, b)

            return carry

        lax.fori_loop(0, _NCHUNK // _NBUF, outer, 0)

    return gather_kernel


def _fm1_body(g_ref, w_ref, o_ref):
    o_ref[...] = jnp.maximum(
        jnp.dot(g_ref[...], w_ref[...], preferred_element_type=jnp.float32),
        0.0,
    )


def _fm1(g1, w1s, interpret=False):
    rows = _B * _N
    blk = 2048
    return pl.pallas_call(
        _fm1_body,
        grid=(rows // blk,),
        in_specs=[
            pl.BlockSpec((blk, _K * _C1P), lambda i: (i, 0)),
            pl.BlockSpec((_K * _C1P, 32), lambda i: (0, 0)),
        ],
        out_specs=pl.BlockSpec((blk, 32), lambda i: (i, 0)),
        out_shape=jax.ShapeDtypeStruct((rows, 32), jnp.float32),
        interpret=interpret,
    )(g1, w1s)


def _fm2_body(g_ref, w_ref, w1_ref, b1_ref, w2_ref, b2_ref, o_ref, acc):
    b = pl.program_id(0)
    h = jnp.maximum(
        jnp.dot(g_ref[...], w_ref[...], preferred_element_type=jnp.float32),
        0.0,
    )
    pool = jnp.sum(h, axis=0, keepdims=True) * (1.0 / _N)
    row = lax.broadcasted_iota(jnp.int32, (_B, 1), 0)
    upd = jnp.where(row == b, pool, 0.0)

    @pl.when(b == 0)
    def _():
        acc[...] = jnp.zeros_like(acc)

    acc[...] += upd

    @pl.when(b == _B - 1)
    def _():
        hh = jnp.maximum(
            jnp.dot(acc[...], w1_ref[...], preferred_element_type=jnp.float32)
            + b1_ref[...],
            0.0,
        )
        o_ref[...] = (
            jnp.dot(hh, w2_ref[...], preferred_element_type=jnp.float32)
            + b2_ref[...]
        )


def _fm2_pool_mlp(g2, w2s, W1, b1, W2, b2, interpret=False):
    return pl.pallas_call(
        _fm2_body,
        grid=(_B,),
        in_specs=[
            pl.BlockSpec((_N, _K * _C2P), lambda b: (b, 0)),
            pl.BlockSpec((_K * _C2P, 40), lambda b: (0, 0)),
            pl.BlockSpec((40, 512), lambda b: (0, 0)),
            pl.BlockSpec((1, 512), lambda b: (0, 0)),
            pl.BlockSpec((512, 40), lambda b: (0, 0)),
            pl.BlockSpec((1, 40), lambda b: (0, 0)),
        ],
        out_specs=pl.BlockSpec((_B, 40), lambda b: (0, 0)),
        out_shape=jax.ShapeDtypeStruct((_B, 40), jnp.float32),
        scratch_shapes=[pltpu.VMEM((_B, 40), jnp.float32)],
        interpret=interpret,
    )(g2, w2s, W1, b1.reshape(1, -1), W2, b2.reshape(1, -1))


@jax.jit
def kernel(inputs, W_fm1, W_fm2, W1, b1, W2, b2):
    w1s, w2s = _prep_weights(W_fm1, W_fm2)
    idx, xpad = _topk(inputs)
    idx3 = idx.reshape(_NW, _NCHUNK, _GCHUNK)

    g1 = _make_sc_gather(_C1P)(xpad, idx3)
    fm1 = _fm1(g1.reshape(_B * _N, _K * _C1P), w1s)
    g2 = _make_sc_gather(_C2P)(fm1, idx3)
    return _fm2_pool_mlp(g2.reshape(_B * _N, _K * _C2P), w2s, W1, b1, W2, b2)


# speedup vs baseline: 1.0548x; 1.0548x over previous
import functools

import jax
import jax.numpy as jnp
from jax import lax
from jax.experimental import pallas as pl
from jax.experimental.pallas import tpu as pltpu
from jax.experimental.pallas import tpu_sc as plsc

_B, _N, _K = 32, 2048, 20
_HB = 16
_C1P = 16
_C2P = 32
_R = 1024

_NW = 32
_GCHUNK = 128
_NBUF = 4
_M = _HB * _N * _K
_MW = _M // _NW
_NCHUNK = _MW // _GCHUNK


def _sel_matrix(cp, c):
    rows = lax.broadcasted_iota(jnp.int32, (_K * cp, _K * c), 0)
    cols = lax.broadcasted_iota(jnp.int32, (_K * cp, _K * c), 1)
    k, r = rows // cp, rows % cp
    return jnp.where((r < c) & (cols == k * c + r), 1.0, 0.0).astype(
        jnp.float32
    )


def _prep_body(w1_ref, w2_ref, o1_ref, o2_ref):
    w1 = w1_ref[...]
    e1 = jnp.exp(w1 - jnp.max(w1, axis=0, keepdims=True))
    s1 = e1 / jnp.sum(e1, axis=0, keepdims=True)
    s1 = jnp.pad(s1, ((0, 0), (0, 2)))
    o1_ref[...] = jnp.dot(
        _sel_matrix(_C1P, 3), s1, preferred_element_type=jnp.float32
    )
    w2 = w2_ref[...]
    e2 = jnp.exp(w2 - jnp.max(w2, axis=0, keepdims=True))
    s2 = e2 / jnp.sum(e2, axis=0, keepdims=True)
    o2_ref[...] = jnp.dot(
        _sel_matrix(_C2P, 30), s2, preferred_element_type=jnp.float32
    )


def _prep_weights(W_fm1, W_fm2, interpret=False):
    return pl.pallas_call(
        _prep_body,
        out_shape=(
            jax.ShapeDtypeStruct((_K * _C1P, 32), jnp.float32),
            jax.ShapeDtypeStruct((_K * _C2P, 40), jnp.float32),
        ),
        interpret=interpret,
    )(W_fm1.reshape(_K * 3, 30), W_fm2.reshape(_K * 30, 40))


def _topk_body(xb_ref, xt_ref, idx_ref, xpad_ref):
    b = pl.program_id(0)
    xb = xb_ref[0]
    xt = xt_ref[0]
    xpad_ref[...] = jnp.pad(xb, ((0, 0), (0, _C1P - 3)))
    sqb = jnp.sum(xb * xb, axis=1, keepdims=True)
    sqf = jnp.sum(xt * xt, axis=0, keepdims=True)
    d = sqb + sqf - 2.0 * jnp.dot(xb, xt, preferred_element_type=jnp.float32)
    col = lax.broadcasted_iota(jnp.int32, (_R, _N), 1)
    key = (
        lax.bitcast_convert_type(jnp.maximum(d, 0.0), jnp.int32) & (-2048)
    ) | col
    int_min = jnp.int32(-(2**31))
    m = jnp.full((_R, 1), -1, jnp.int32)
    cols = []
    for _ in range(_K):
        s = m + (int_min + 1)
        v = key - s
        m = jnp.min(v, axis=1, keepdims=True) + s
        cols.append(m & 2047)
    idx_ref[0] = jnp.concatenate(cols, axis=1) + b * _N


def _topk(inputs, interpret=False):
    nb = inputs.shape[0]
    xt = jnp.transpose(inputs, (0, 2, 1))
    return pl.pallas_call(
        _topk_body,
        grid=(nb, _N // _R),
        in_specs=[
            pl.BlockSpec((1, _R, 3), lambda b, r: (b, r, 0)),
            pl.BlockSpec((1, 3, _N), lambda b, r: (b, 0, 0)),
        ],
        out_specs=[
            pl.BlockSpec((1, _R, _K), lambda b, r: (b, r, 0)),
            pl.BlockSpec((_R, _C1P), lambda b, r: (b * (_N // _R) + r, 0)),
        ],
        out_shape=[
            jax.ShapeDtypeStruct((nb, _N, _K), jnp.int32),
            jax.ShapeDtypeStruct((nb * _N, _C1P), jnp.float32),
        ],
        interpret=interpret,
    )(inputs, xt)


@functools.lru_cache(maxsize=None)
def _make_sc_gather(width):
    mesh = plsc.VectorSubcoreMesh(core_axis_name="c", subcore_axis_name="s")

    @functools.partial(
        pl.kernel,
        out_type=jax.ShapeDtypeStruct((_M, width), jnp.float32),
        mesh=mesh,
        scratch_types=[
            pltpu.VMEM((_NCHUNK, _GCHUNK), jnp.int32),
            pltpu.VMEM((_NBUF, _GCHUNK, width), jnp.float32),
        ] + [pltpu.SemaphoreType.DMA] * _NBUF,
        compiler_params=pltpu.CompilerParams(use_tc_tiling_on_sc=False),
    )
    def gather_kernel(table_hbm, idx_hbm, out_hbm, idxv, bufv, *sems):
        c = lax.axis_index("c")
        s = lax.axis_index("s")
        wid = s * 2 + c
        base = wid * _MW
        pltpu.sync_copy(idx_hbm.at[wid], idxv)

        def fire(chunk, b):
            pltpu.async_copy(table_hbm.at[idxv.at[chunk]], bufv.at[b], sems[b])

        def wait(b):
            pltpu.make_async_copy(
                table_hbm.at[idxv.at[0]], bufv.at[b], sems[b]
            ).wait()

        for b in range(_NBUF):
            fire(b, b)

        def outer(g, carry):
            for b in range(_NBUF):
                chunk = g * _NBUF + b
                wait(b)
                pltpu.sync_copy(
                    bufv.at[b],
                    out_hbm.at[pl.ds(base + chunk * _GCHUNK, _GCHUNK)],
                )

                @pl.when(chunk + _NBUF < _NCHUNK)
                def _():
                    fire(chunk + _NBUF, b)

            return carry

        lax.fori_loop(0, _NCHUNK // _NBUF, outer, 0)

    return gather_kernel


def _fm1_body(g_ref, w_ref, o_ref):
    o_ref[...] = jnp.maximum(
        jnp.dot(g_ref[...], w_ref[...], preferred_element_type=jnp.float32),
        0.0,
    )


def _fm1(g1, w1s, interpret=False):
    rows = _HB * _N
    blk = 2048
    return pl.pallas_call(
        _fm1_body,
        grid=(rows // blk,),
        in_specs=[
            pl.BlockSpec((blk, _K * _C1P), lambda i: (i, 0)),
            pl.BlockSpec((_K * _C1P, 32), lambda i: (0, 0)),
        ],
        out_specs=pl.BlockSpec((blk, 32), lambda i: (i, 0)),
        out_shape=jax.ShapeDtypeStruct((rows, 32), jnp.float32),
        interpret=interpret,
    )(g1, w1s)


def _fm2_body(g_ref, w_ref, o_ref, acc):
    b = pl.program_id(0)
    h = jnp.maximum(
        jnp.dot(g_ref[...], w_ref[...], preferred_element_type=jnp.float32),
        0.0,
    )
    pool = jnp.sum(h, axis=0, keepdims=True) * (1.0 / _N)
    row = lax.broadcasted_iota(jnp.int32, (_HB, 1), 0)
    upd = jnp.where(row == b, pool, 0.0)

    @pl.when(b == 0)
    def _():
        acc[...] = jnp.zeros_like(acc)

    acc[...] += upd

    @pl.when(b == _HB - 1)
    def _():
        o_ref[...] = acc[...]


def _fm2_pool(g2, w2s, interpret=False):
    return pl.pallas_call(
        _fm2_body,
        grid=(_HB,),
        in_specs=[
            pl.BlockSpec((_N, _K * _C2P), lambda b: (b, 0)),
            pl.BlockSpec((_K * _C2P, 40), lambda b: (0, 0)),
        ],
        out_specs=pl.BlockSpec((_HB, 40), lambda b: (0, 0)),
        out_shape=jax.ShapeDtypeStruct((_HB, 40), jnp.float32),
        scratch_shapes=[pltpu.VMEM((_HB, 40), jnp.float32)],
        interpret=interpret,
    )(g2, w2s)


def _mlp_body(p1_ref, p2_ref, w1_ref, b1_ref, w2_ref, b2_ref, o_ref):
    p = jnp.concatenate([p1_ref[...], p2_ref[...]], axis=0)
    h = jnp.maximum(
        jnp.dot(p, w1_ref[...], preferred_element_type=jnp.float32)
        + b1_ref[...],
        0.0,
    )
    o_ref[...] = (
        jnp.dot(h, w2_ref[...], preferred_element_type=jnp.float32)
        + b2_ref[...]
    )


def _mlp(p1, p2, W1, b1, W2, b2, interpret=False):
    return pl.pallas_call(
        _mlp_body,
        out_shape=jax.ShapeDtypeStruct((_B, 40), jnp.float32),
        interpret=interpret,
    )(p1, p2, W1, b1.reshape(1, -1), W2, b2.reshape(1, -1))


def _half(xh, w1s, w2s):
    idx, xpad = _topk(xh)
    idx3 = idx.reshape(_NW, _NCHUNK, _GCHUNK)
    g1 = _make_sc_gather(_C1P)(xpad, idx3)
    fm1 = _fm1(g1.reshape(_HB * _N, _K * _C1P), w1s)
    g2 = _make_sc_gather(_C2P)(fm1, idx3)
    return _fm2_pool(g2.reshape(_HB * _N, _K * _C2P), w2s)


@jax.jit
def kernel(inputs, W_fm1, W_fm2, W1, b1, W2, b2):
    w1s, w2s = _prep_weights(W_fm1, W_fm2)
    p1 = _half(inputs[:_HB], w1s, w2s)
    p2 = _half(inputs[_HB:], w1s, w2s)
    return _mlp(p1, p2, W1, b1, W2, b2)
